# baseline (speedup 1.0000x reference)
import functools
import jax
import jax.numpy as jnp
from jax import lax
from jax.experimental import pallas as pl
from jax.experimental.pallas import tpu as pltpu
from jax.experimental.pallas import tpu_sc as plsc

_B = 256
_D = 256
_STEPS = 4
_NW = 32
_C = 1600
_NP = _NW * _C
_SUB = 64
_NSUB = _C // _SUB
_BP = 272
_QROWS = 264
_HI = jax.lax.Precision.HIGHEST
_NEG_INF = float("-inf")

def _sc_body(x_hbm, b_hbm, q_hbm, m_hbm, s_hbm, v_hbm,
             xbuf, qrows, idxbuf, mloc, sloc, vloc, hbuf, sem):
    wid = lax.axis_index("s") * 2 + lax.axis_index("c")
    base = wid * _C

    zeros16 = jnp.zeros((16,), jnp.float32)
    neginf16 = jnp.full((16,), _NEG_INF, jnp.float32)
    iota16 = jax.lax.iota(jnp.int32, 16)
    lane0 = iota16 == 0
    rot = [jax.lax.bitwise_and(iota16 + k, 15) for k in (8, 4, 2, 1)]

    def hsum16(x):
        for idx in rot:
            hbuf[...] = x
            x = x + plsc.load_gather(hbuf, [idx])
        return x
    for i in range(_BP // 16):
        mloc[pl.ds(16 * i, 16)] = neginf16
        sloc[pl.ds(16 * i, 16)] = zeros16

    def zero_v(i, carry):
        vloc[pl.ds(16 * i, 16)] = zeros16
        return carry

    lax.fori_loop(0, _BP * _D // 16, zero_v, 0)

    def node(ng, b_n, carry):
        cur_b, cur_m16, cur_s16 = carry

        acc = zeros16
        for k in range(16):
            acc = acc + xbuf[ng, pl.ds(16 * k, 16)] * qrows[ng, pl.ds(16 * k, 16)]
        e16 = hsum16(acc)

        b16 = lax.broadcast(b_n, (16,))
        fresh16 = b16 != lax.broadcast(cur_b, (16,))
        cur_m16 = jnp.where(fresh16, neginf16, cur_m16)
        cur_s16 = jnp.where(fresh16, zeros16, cur_s16)

        new_m16 = jnp.maximum(cur_m16, e16)
        ratio16 = jnp.exp(cur_m16 - new_m16)
        w16 = jnp.exp(e16 - new_m16)
        new_s16 = cur_s16 * ratio16 + w16

        off = b_n * _D
        for k in range(16):
            sl = pl.ds(off + 16 * k, 16)
            vloc[sl] = vloc[sl] * ratio16 + w16 * xbuf[ng, pl.ds(16 * k, 16)]

        plsc.store_scatter(mloc, [b16], new_m16, mask=lane0)
        plsc.store_scatter(sloc, [b16], new_s16, mask=lane0)
        return (b_n, new_m16, new_s16)

    def group(g, carry):
        bv16 = idxbuf[pl.ds(16 * g, 16)]
        for l in range(16):
            carry = node(16 * g + l, bv16[l], carry)
        return carry

    def subchunk(j, carry):
        row0 = base + j * _SUB
        pltpu.sync_copy(x_hbm.at[pl.ds(row0, _SUB), :], xbuf)
        pltpu.sync_copy(b_hbm.at[pl.ds(row0, _SUB)], idxbuf)
        pltpu.async_copy(q_hbm.at[idxbuf], qrows, sem).wait()
        return lax.fori_loop(0, _SUB // 16, group, carry)

    lax.fori_loop(0, _NSUB, subchunk,
                  (jnp.int32(-1), neginf16, zeros16))

    pltpu.sync_copy(mloc, m_hbm.at[wid])
    pltpu.sync_copy(sloc, s_hbm.at[wid])
    pltpu.sync_copy(vloc, v_hbm.at[wid])


def _sc_attn(x_pad, b_pad, q_pad):
    mesh = plsc.VectorSubcoreMesh(core_axis_name="c", subcore_axis_name="s")
    fn = functools.partial(
        pl.kernel, mesh=mesh,
        out_type=[
            jax.ShapeDtypeStruct((_NW, _BP), jnp.float32),
            jax.ShapeDtypeStruct((_NW, _BP), jnp.float32),
            jax.ShapeDtypeStruct((_NW, _BP * _D), jnp.float32),
        ],
        scratch_types=[
            pltpu.VMEM((_SUB, _D), jnp.float32),
            pltpu.VMEM((_SUB, _D), jnp.float32),
            pltpu.VMEM((_SUB,), jnp.int32),
            pltpu.VMEM((_BP,), jnp.float32),
            pltpu.VMEM((_BP,), jnp.float32),
            pltpu.VMEM((_BP * _D,), jnp.float32),
            pltpu.VMEM((16,), jnp.float32),
            pltpu.SemaphoreType.DMA,
        ],
        compiler_params=pltpu.CompilerParams(needs_layout_passes=False),
    )(_sc_body)
    return fn(x_pad, b_pad, q_pad)


def _tc_body(m_ref, s_ref, v_ref, qprev_ref, h_ref, c_ref,
             wih_ref, whh_ref, bias_ref,
             qstar_ref, qn_ref, hn_ref, cn_ref,
             mg, sg, rnum):
    t = pl.program_id(0)
    nt = pl.num_programs(0)

    @pl.when(t == 0)
    def _first():
        mg[0, :] = jnp.max(m_ref[...], axis=0)
        sg[...] = jnp.zeros_like(sg)
        rnum[...] = jnp.zeros_like(rnum)

    mt = m_ref[t, :]
    wfac = jnp.where(mt == _NEG_INF, 0.0, jnp.exp(mt - mg[0, :]))
    sg[0, :] = sg[0, :] + wfac * s_ref[t, :]
    vt = v_ref[0, :, :].reshape(_BP, _D)
    rnum[...] = rnum[...] + wfac[0:_B, None] * vt[0:_B, :]

    @pl.when(t == nt - 1)
    def _last():
        r = rnum[...] / (sg[0, 0:_B][:, None] + 1e-16)
        qstar_ref[:, 0:_D] = qprev_ref[...]
        qstar_ref[:, _D:2 * _D] = r
        gates = (jnp.dot(qstar_ref[...], wih_ref[...], precision=_HI)
                 + jnp.dot(h_ref[...], whh_ref[...], precision=_HI)
                 + bias_ref[...])
        i = jax.nn.sigmoid(gates[:, 0:_D])
        f = jax.nn.sigmoid(gates[:, _D:2 * _D])
        g = jnp.tanh(gates[:, 2 * _D:3 * _D])
        o = jax.nn.sigmoid(gates[:, 3 * _D:4 * _D])
        cc = f * c_ref[...] + i * g
        cn_ref[...] = cc
        hh = o * jnp.tanh(cc)
        hn_ref[...] = hh
        qn_ref[...] = hh


def _tc_combine_lstm(m_t, s_t, v_t, qprev, h, c, wiht, whht, bias):
    v3 = v_t.reshape(_NW, _BP, _D)
    return pl.pallas_call(
        _tc_body,
        grid=(_NW,),
        in_specs=[
            pl.BlockSpec((_NW, _BP), lambda t: (0, 0)),
            pl.BlockSpec((_NW, _BP), lambda t: (0, 0)),
            pl.BlockSpec((1, _BP, _D), lambda t: (t, 0, 0)),
            pl.BlockSpec((_B, _D), lambda t: (0, 0)),
            pl.BlockSpec((_B, _D), lambda t: (0, 0)),
            pl.BlockSpec((_B, _D), lambda t: (0, 0)),
            pl.BlockSpec((2 * _D, 4 * _D), lambda t: (0, 0)),
            pl.BlockSpec((_D, 4 * _D), lambda t: (0, 0)),
            pl.BlockSpec((1, 4 * _D), lambda t: (0, 0)),
        ],
        out_specs=[
            pl.BlockSpec((_B, 2 * _D), lambda t: (0, 0)),
            pl.BlockSpec((_B, _D), lambda t: (0, 0)),
            pl.BlockSpec((_B, _D), lambda t: (0, 0)),
            pl.BlockSpec((_B, _D), lambda t: (0, 0)),
        ],
        out_shape=[
            jax.ShapeDtypeStruct((_B, 2 * _D), jnp.float32),
            jax.ShapeDtypeStruct((_B, _D), jnp.float32),
            jax.ShapeDtypeStruct((_B, _D), jnp.float32),
            jax.ShapeDtypeStruct((_B, _D), jnp.float32),
        ],
        scratch_shapes=[
            pltpu.VMEM((1, _BP), jnp.float32),
            pltpu.VMEM((1, _BP), jnp.float32),
            pltpu.VMEM((_B, _D), jnp.float32),
        ],
        compiler_params=pltpu.CompilerParams(
            dimension_semantics=("arbitrary",)),
    )(m_t, s_t, v3, qprev, h, c, wiht, whht, bias)


def kernel(x, batch, W_ih, W_hh, b_ih, b_hh):
    n = x.shape[0]
    batch = batch.astype(jnp.int32)
    x_pad = jnp.pad(x, ((0, _NP - n), (0, 0)))
    b_pad = jnp.pad(batch, (0, _NP - n), constant_values=_B)
    wiht = W_ih.T
    whht = W_hh.T
    bias = (b_ih + b_hh).reshape(1, 4 * _D).astype(jnp.float32)

    m0 = jnp.full((_NW, _BP), _NEG_INF, jnp.float32)
    s0 = jnp.zeros((_NW, _BP), jnp.float32)
    v0 = jnp.zeros((_NW, _BP * _D), jnp.float32)
    zq = jnp.zeros((_B, _D), jnp.float32)

    qstar, q, h, c = _tc_combine_lstm(m0, s0, v0, zq, zq, zq,
                                      wiht, whht, bias)
    for _step in range(_STEPS):
        q_pad = jnp.pad(q, ((0, _QROWS - _B), (0, 0)))
        m_t, s_t, v_t = _sc_attn(x_pad, b_pad, q_pad)
        qstar, q, h, c = _tc_combine_lstm(m_t, s_t, v_t, q, h, c,
                                          wiht, whht, bias)
    return qstar


# speedup vs baseline: 7.1075x; 7.1075x over previous
import jax
import jax.numpy as jnp
from jax.experimental import pallas as pl
from jax.experimental.pallas import tpu as pltpu

_B = 256
_D = 256
_STEPS = 4
_TILE = 2048
_HI = jax.lax.Precision.HIGHEST


_DN_T = (((0,), (0,)), ((), ()))


def _split_dot(a_bf16, b_f32, dn=None):
    b_hi = b_f32.astype(jnp.bfloat16)
    b_lo = (b_f32 - b_hi.astype(jnp.float32)).astype(jnp.bfloat16)
    if dn is None:
        hi = jax.lax.dot(a_bf16, b_hi, preferred_element_type=jnp.float32)
        lo = jax.lax.dot(a_bf16, b_lo, preferred_element_type=jnp.float32)
    else:
        hi = jax.lax.dot_general(a_bf16, b_hi, dn,
                                 preferred_element_type=jnp.float32)
        lo = jax.lax.dot_general(a_bf16, b_lo, dn,
                                 preferred_element_type=jnp.float32)
    return hi + lo


def _body(x_ref, b_ref, wih_ref, whh_ref, bias_ref, out_ref,
          qstar, h, c, q, m, s_acc, r_acc):
    st = pl.program_id(0)
    t = pl.program_id(1)
    num_tiles = pl.num_programs(1)

    @pl.when(jnp.logical_and(st == 0, t == 0))
    def _init():
        qstar[...] = jnp.zeros_like(qstar)
        h[...] = jnp.zeros_like(h)
        c[...] = jnp.zeros_like(c)

    @pl.when(t == 0)
    def _lstm():
        gates = (jnp.dot(qstar[...], wih_ref[...], precision=_HI)
                 + jnp.dot(h[...], whh_ref[...], precision=_HI)
                 + bias_ref[...])
        i = jax.nn.sigmoid(gates[:, 0:_D])
        f = jax.nn.sigmoid(gates[:, _D:2 * _D])
        g = jnp.tanh(gates[:, 2 * _D:3 * _D])
        o = jax.nn.sigmoid(gates[:, 3 * _D:4 * _D])
        cc = f * c[...] + i * g
        c[...] = cc
        hh = o * jnp.tanh(cc)
        h[...] = hh
        q[...] = hh
        m[...] = jnp.full_like(m, -jnp.inf)
        s_acc[...] = jnp.zeros_like(s_acc)
        r_acc[...] = jnp.zeros_like(r_acc)

    ids = b_ref[0, 0, :]
    cols = jax.lax.broadcasted_iota(jnp.int32, (_TILE, _B), 1)
    onehot_b = (ids[:, None] == cols)
    onehot = onehot_b.astype(jnp.float32)
    onehot_bf = onehot.astype(jnp.bfloat16)

    qg = _split_dot(onehot_bf, q[...])
    x = x_ref[...]
    e = jnp.sum(x * qg, axis=1)

    m_old = m[0, :]
    tile_max = jnp.max(jnp.where(onehot_b, e[:, None], -jnp.inf), axis=0)
    m_new = jnp.maximum(m_old, tile_max)
    m[0, :] = m_new
    scale = jnp.where(m_new == -jnp.inf, 1.0, jnp.exp(m_old - m_new))

    mg = jnp.sum(onehot * m_new[None, :], axis=1)
    ee = jnp.exp(e - mg)
    s_acc[0, :] = s_acc[0, :] * scale + jnp.sum(onehot * ee[:, None], axis=0)
    wx = ee[:, None] * x
    r_acc[...] = (r_acc[...] * scale[:, None]
                  + _split_dot(onehot_bf, wx, dn=_DN_T))

    @pl.when(t == num_tiles - 1)
    def _finish():
        r = r_acc[...] / (s_acc[0, :][:, None] + 1e-16)
        qstar[:, 0:_D] = q[...]
        qstar[:, _D:2 * _D] = r

        @pl.when(st == _STEPS - 1)
        def _out():
            out_ref[...] = qstar[...]


def _set2set_tc(x, batch_i32, w_iht, w_hht, bias):
    n = x.shape[0]
    num_tiles = pl.cdiv(n, _TILE)
    n_pad = num_tiles * _TILE
    x_pad = jnp.pad(x, ((0, n_pad - n), (0, 0)))
    b_pad = jnp.pad(batch_i32, (0, n_pad - n), constant_values=_B)
    b3 = b_pad.reshape(num_tiles, 1, _TILE)

    return pl.pallas_call(
        _body,
        grid=(_STEPS, num_tiles),
        in_specs=[
            pl.BlockSpec((_TILE, _D), lambda s, t: (t, 0)),
            pl.BlockSpec((1, 1, _TILE), lambda s, t: (t, 0, 0)),
            pl.BlockSpec((2 * _D, 4 * _D), lambda s, t: (0, 0)),
            pl.BlockSpec((_D, 4 * _D), lambda s, t: (0, 0)),
            pl.BlockSpec((1, 4 * _D), lambda s, t: (0, 0)),
        ],
        out_specs=pl.BlockSpec((_B, 2 * _D), lambda s, t: (0, 0)),
        out_shape=jax.ShapeDtypeStruct((_B, 2 * _D), jnp.float32),
        scratch_shapes=[
            pltpu.VMEM((_B, 2 * _D), jnp.float32),
            pltpu.VMEM((_B, _D), jnp.float32),
            pltpu.VMEM((_B, _D), jnp.float32),
            pltpu.VMEM((_B, _D), jnp.float32),
            pltpu.VMEM((1, _B), jnp.float32),
            pltpu.VMEM((1, _B), jnp.float32),
            pltpu.VMEM((_B, _D), jnp.float32),
        ],
        compiler_params=pltpu.CompilerParams(
            dimension_semantics=("arbitrary", "arbitrary")),
    )(x_pad, b3, w_iht, w_hht, bias)


def kernel(x, batch, W_ih, W_hh, b_ih, b_hh):
    batch = batch.astype(jnp.int32)
    bias = (b_ih + b_hh).reshape(1, 4 * _D).astype(jnp.float32)
    return _set2set_tc(x, batch, W_ih.T, W_hh.T, bias)


# speedup vs baseline: 11.5282x; 1.6220x over previous
import jax
import jax.numpy as jnp
from jax.experimental import pallas as pl
from jax.experimental.pallas import tpu as pltpu

_B = 256
_D = 256
_STEPS = 4
_TILE = 4096
_HI = jax.lax.Precision.HIGHEST


_DN_T = (((0,), (0,)), ((), ()))


def _split_dot(a_bf16, b_f32, dn=None):
    b_hi = b_f32.astype(jnp.bfloat16)
    b_lo = (b_f32 - b_hi.astype(jnp.float32)).astype(jnp.bfloat16)
    if dn is None:
        hi = jax.lax.dot(a_bf16, b_hi, preferred_element_type=jnp.float32)
        lo = jax.lax.dot(a_bf16, b_lo, preferred_element_type=jnp.float32)
    else:
        hi = jax.lax.dot_general(a_bf16, b_hi, dn,
                                 preferred_element_type=jnp.float32)
        lo = jax.lax.dot_general(a_bf16, b_lo, dn,
                                 preferred_element_type=jnp.float32)
    return hi + lo


def _body(x_ref, b_ref, wih_ref, whh_ref, bias_ref, out_ref,
          qstar, h, c, q, m, s_acc, r_acc):
    st = pl.program_id(0)
    t = pl.program_id(1)
    num_tiles = pl.num_programs(1)

    @pl.when(jnp.logical_and(st == 0, t == 0))
    def _init():
        qstar[...] = jnp.zeros_like(qstar)
        h[...] = jnp.zeros_like(h)
        c[...] = jnp.zeros_like(c)

    @pl.when(t == 0)
    def _lstm():
        gates = (jnp.dot(qstar[...], wih_ref[...], precision=_HI)
                 + jnp.dot(h[...], whh_ref[...], precision=_HI)
                 + bias_ref[...])
        i = jax.nn.sigmoid(gates[:, 0:_D])
        f = jax.nn.sigmoid(gates[:, _D:2 * _D])
        g = jnp.tanh(gates[:, 2 * _D:3 * _D])
        o = jax.nn.sigmoid(gates[:, 3 * _D:4 * _D])
        cc = f * c[...] + i * g
        c[...] = cc
        hh = o * jnp.tanh(cc)
        h[...] = hh
        q[...] = hh
        m[...] = jnp.full_like(m, -jnp.inf)
        s_acc[...] = jnp.zeros_like(s_acc)
        r_acc[...] = jnp.zeros_like(r_acc)

    ids = b_ref[0, 0, :]
    cols = jax.lax.broadcasted_iota(jnp.int32, (_TILE, _B), 1)
    onehot_b = (ids[:, None] == cols)
    onehot_bf = onehot_b.astype(jnp.bfloat16)

    qg = _split_dot(onehot_bf, q[...])
    x = x_ref[...]
    e = jnp.sum(x * qg, axis=1)

    m_old = m[0, :]
    tile_max = jnp.max(jnp.where(onehot_b, e[:, None], -jnp.inf), axis=0)
    m_new = jnp.maximum(m_old, tile_max)
    m[0, :] = m_new
    scale = jnp.where(m_new == -jnp.inf, 1.0, jnp.exp(m_old - m_new))

    mg = jnp.sum(jnp.where(onehot_b, m_new[None, :], 0.0), axis=1)
    ee = jnp.exp(e - mg)
    s_acc[0, :] = (s_acc[0, :] * scale
                   + jnp.sum(jnp.where(onehot_b, ee[:, None], 0.0), axis=0))
    wx = ee[:, None] * x
    r_acc[...] = (r_acc[...] * scale[:, None]
                  + _split_dot(onehot_bf, wx, dn=_DN_T))

    @pl.when(t == num_tiles - 1)
    def _finish():
        r = r_acc[...] / (s_acc[0, :][:, None] + 1e-16)
        qstar[:, 0:_D] = q[...]
        qstar[:, _D:2 * _D] = r

        @pl.when(st == _STEPS - 1)
        def _out():
            out_ref[...] = qstar[...]


def _set2set_tc(x, batch_i32, w_iht, w_hht, bias):
    n = x.shape[0]
    num_tiles = pl.cdiv(n, _TILE)
    n_pad = num_tiles * _TILE
    x_pad = jnp.pad(x, ((0, n_pad - n), (0, 0)))
    b_pad = jnp.pad(batch_i32, (0, n_pad - n), constant_values=_B)
    b3 = b_pad.reshape(num_tiles, 1, _TILE)

    return pl.pallas_call(
        _body,
        grid=(_STEPS, num_tiles),
        in_specs=[
            pl.BlockSpec((_TILE, _D), lambda s, t: (t, 0)),
            pl.BlockSpec((1, 1, _TILE), lambda s, t: (t, 0, 0)),
            pl.BlockSpec((2 * _D, 4 * _D), lambda s, t: (0, 0)),
            pl.BlockSpec((_D, 4 * _D), lambda s, t: (0, 0)),
            pl.BlockSpec((1, 4 * _D), lambda s, t: (0, 0)),
        ],
        out_specs=pl.BlockSpec((_B, 2 * _D), lambda s, t: (0, 0)),
        out_shape=jax.ShapeDtypeStruct((_B, 2 * _D), jnp.float32),
        scratch_shapes=[
            pltpu.VMEM((_B, 2 * _D), jnp.float32),
            pltpu.VMEM((_B, _D), jnp.float32),
            pltpu.VMEM((_B, _D), jnp.float32),
            pltpu.VMEM((_B, _D), jnp.float32),
            pltpu.VMEM((1, _B), jnp.float32),
            pltpu.VMEM((1, _B), jnp.float32),
            pltpu.VMEM((_B, _D), jnp.float32),
        ],
        compiler_params=pltpu.CompilerParams(
            dimension_semantics=("arbitrary", "arbitrary")),
    )(x_pad, b3, w_iht, w_hht, bias)


def kernel(x, batch, W_ih, W_hh, b_ih, b_hh):
    batch = batch.astype(jnp.int32)
    bias = (b_ih + b_hh).reshape(1, 4 * _D).astype(jnp.float32)
    return _set2set_tc(x, batch, W_ih.T, W_hh.T, bias)
